# baseline (speedup 1.0000x reference)
import jax
import jax.numpy as jnp
from jax import lax
from jax.experimental import pallas as pl
from jax.experimental.pallas import tpu as pltpu

_NUM_NETS = 8
_NUM_PROXIES = 100000
_EMBED_DIM = 16
_B = 26
_WIN = 128
_H = 2
_CHUNK = 51200


def _body(scond_ref, perm_ref, sadj_ref, t_ref, win_ref, z_ref, x_ref):
    h = pl.program_id(0)
    i = pl.program_id(1)
    z_ref[...] = t_ref[...]

    @pl.when((i == 0) & (h == 0))
    def _():
        x_ref[...] = jnp.zeros((_EMBED_DIM, _B), jnp.float32)

    @pl.when(h == 0)
    def _():
        a = sadj_ref[i]
        b = perm_ref[i]
        lane = a % _WIN
        colmask = (
            lax.broadcasted_iota(jnp.int32, (_EMBED_DIM, _WIN), 1) == lane
        )
        col = jnp.sum(
            jnp.where(colmask, win_ref[...], 0.0), axis=1, keepdims=True
        )
        bmask = lax.broadcasted_iota(jnp.int32, (_EMBED_DIM, _B), 1) == b
        x_ref[...] = x_ref[...] + jnp.where(bmask, col, 0.0)


def kernel(tables, cond, adjs):
    perm = jnp.argsort(cond).astype(jnp.int32)
    scond = cond[perm]
    sadj = adjs[perm]

    tt = jnp.transpose(tables, (0, 2, 1))

    grid_spec = pltpu.PrefetchScalarGridSpec(
        num_scalar_prefetch=3,
        grid=(_H, _B),
        in_specs=[
            pl.BlockSpec(
                (None, _EMBED_DIM, _CHUNK),
                lambda h, i, sc, pm, sa: (sc[i], 0, h),
            ),
            pl.BlockSpec(
                (None, _EMBED_DIM, _WIN),
                lambda h, i, sc, pm, sa: (sc[i], 0, sa[i] // _WIN),
            ),
        ],
        out_specs=[
            pl.BlockSpec(
                (None, _EMBED_DIM, _CHUNK),
                lambda h, i, sc, pm, sa: (pm[i], 0, h),
            ),
            pl.BlockSpec((_EMBED_DIM, _B), lambda h, i, sc, pm, sa: (0, 0)),
        ],
    )

    zt, xt = pl.pallas_call(
        _body,
        grid_spec=grid_spec,
        out_shape=[
            jax.ShapeDtypeStruct((_B, _EMBED_DIM, _NUM_PROXIES), jnp.float32),
            jax.ShapeDtypeStruct((_EMBED_DIM, _B), jnp.float32),
        ],
        compiler_params=pltpu.CompilerParams(
            dimension_semantics=("arbitrary", "arbitrary"),
        ),
    )(scond, perm, sadj, tt, tt)

    z = jnp.transpose(zt, (0, 2, 1))
    x = xt.T
    return (x, z)


# speedup vs baseline: 1.1250x; 1.1250x over previous
import jax
import jax.numpy as jnp
from jax import lax
from jax.experimental import pallas as pl
from jax.experimental.pallas import tpu as pltpu

_NUM_NETS = 8
_NUM_PROXIES = 100000
_EMBED_DIM = 16
_B = 26
_WIN = 128
_H = 1
_CHUNK = _NUM_PROXIES


def _body(scond_ref, perm_ref, sadj_ref, t_ref, win_ref, z_ref, x_ref):
    h = pl.program_id(0)
    i = pl.program_id(1)
    z_ref[...] = t_ref[...]

    @pl.when((i == 0) & (h == 0))
    def _():
        x_ref[...] = jnp.zeros((_EMBED_DIM, _B), jnp.float32)

    @pl.when(h == 0)
    def _():
        a = sadj_ref[i]
        b = perm_ref[i]
        lane = a % _WIN
        colmask = (
            lax.broadcasted_iota(jnp.int32, (_EMBED_DIM, _WIN), 1) == lane
        )
        col = jnp.sum(
            jnp.where(colmask, win_ref[...], 0.0), axis=1, keepdims=True
        )
        bmask = lax.broadcasted_iota(jnp.int32, (_EMBED_DIM, _B), 1) == b
        x_ref[...] = x_ref[...] + jnp.where(bmask, col, 0.0)


def kernel(tables, cond, adjs):
    perm = jnp.argsort(cond).astype(jnp.int32)
    scond = cond[perm]
    sadj = adjs[perm]

    tt = jnp.transpose(tables, (0, 2, 1))

    grid_spec = pltpu.PrefetchScalarGridSpec(
        num_scalar_prefetch=3,
        grid=(_H, _B),
        in_specs=[
            pl.BlockSpec(
                (None, _EMBED_DIM, _CHUNK),
                lambda h, i, sc, pm, sa: (sc[i], 0, h),
            ),
            pl.BlockSpec(
                (None, _EMBED_DIM, _WIN),
                lambda h, i, sc, pm, sa: (sc[i], 0, sa[i] // _WIN),
            ),
        ],
        out_specs=[
            pl.BlockSpec(
                (None, _EMBED_DIM, _CHUNK),
                lambda h, i, sc, pm, sa: (pm[i], 0, h),
            ),
            pl.BlockSpec((_EMBED_DIM, _B), lambda h, i, sc, pm, sa: (0, 0)),
        ],
    )

    zt, xt = pl.pallas_call(
        _body,
        grid_spec=grid_spec,
        out_shape=[
            jax.ShapeDtypeStruct((_B, _EMBED_DIM, _NUM_PROXIES), jnp.float32),
            jax.ShapeDtypeStruct((_EMBED_DIM, _B), jnp.float32),
        ],
        compiler_params=pltpu.CompilerParams(
            dimension_semantics=("arbitrary", "arbitrary"),
        ),
    )(scond, perm, sadj, tt, tt)

    z = jnp.transpose(zt, (0, 2, 1))
    x = xt.T
    return (x, z)
